# baseline (speedup 1.0000x reference)
import functools

import jax
import jax.numpy as jnp
from jax import lax
from jax.experimental import pallas as pl
from jax.experimental.pallas import tpu as pltpu
from jax.experimental.pallas import tpu_sc as plsc

NUM_STYLES = 100000
DIM = 64
BATCH = 16384
P_DROP = 0.5

_info = plsc.get_sparse_core_info()
_NC, _NS, _L = _info.num_cores, _info.num_subcores, _info.num_lanes
_NW = _NC * _NS
_BPW = BATCH // _NW
_GCH = 128
_NG = _BPW // _GCH
_NTRASH = 512

_mesh = plsc.VectorSubcoreMesh(core_axis_name="c", subcore_axis_name="s")


@functools.partial(
    pl.kernel,
    mesh=_mesh,
    compiler_params=pltpu.CompilerParams(use_tc_tiling_on_sc=False),
    out_type=jax.ShapeDtypeStruct((BATCH + _NTRASH, DIM), jnp.float32),
    scratch_types=[
        pltpu.VMEM((_NG, _GCH), jnp.int32),
        pltpu.VMEM((_NG, _GCH), jnp.int32),
        pltpu.VMEM((_BPW,), jnp.int32),
        pltpu.VMEM((_BPW,), jnp.float32),
        pltpu.VMEM((_GCH, DIM), jnp.float32),
        pltpu.VMEM((_BPW, DIM), jnp.float32),
        pltpu.SemaphoreType.DMA,
        pltpu.SemaphoreType.DMA,
    ],
)
def _style_embed(style_hbm, domain_hbm, u_hbm, emb_hbm, out_hbm,
                 gid_v, fix_v, dom_v, u_v, rep0_v, slab_v, sem, sem2):
    wid = lax.axis_index("s") * _NC + lax.axis_index("c")
    base = wid * _BPW
    stages = [
        pltpu.async_copy(style_hbm.at[pl.ds(base + j * _GCH, _GCH)],
                         gid_v.at[j], sem2)
        for j in range(_NG)
    ]
    pltpu.sync_copy(domain_hbm.at[pl.ds(base, _BPW)], dom_v)
    pltpu.sync_copy(u_hbm.at[pl.ds(base, _BPW)], u_v)
    for st in stages:
        st.wait()
    gathers = [
        pltpu.async_copy(emb_hbm.at[gid_v.at[j]],
                         slab_v.at[pl.ds(j * _GCH, _GCH)], sem)
        for j in range(_NG)
    ]
    iota = lax.iota(jnp.int32, _L)
    for g in range(_BPW // _L):
        off = g * _L
        d = dom_v[pl.ds(off, _L)]
        u = u_v[pl.ds(off, _L)]
        drop = (d == 1) & (u < P_DROP)
        lane = iota + off
        trash = BATCH + ((base + off) % _NTRASH) + iota
        fix_v[off // _GCH, pl.ds(off % _GCH, _L)] = jnp.where(
            drop, base + lane, trash)
    pltpu.sync_copy(emb_hbm.at[pl.ds(0, 1)], rep0_v.at[pl.ds(0, 1)])
    r0 = [rep0_v[0, pl.ds(c * _L, _L)] for c in range(DIM // _L)]
    for r in range(1, _GCH):
        for c in range(DIM // _L):
            rep0_v[r, pl.ds(c * _L, _L)] = r0[c]
    for cp in gathers:
        cp.wait()
    pltpu.sync_copy(slab_v, out_hbm.at[pl.ds(base, _BPW)])
    fixups = [
        pltpu.async_copy(rep0_v, out_hbm.at[fix_v.at[j]], sem)
        for j in range(_NG)
    ]
    for cp in fixups:
        cp.wait()


def kernel(style_id, domain_id, emb):
    u = jax.random.uniform(jax.random.key(42), style_id.shape, dtype=jnp.float32)
    out = _style_embed(style_id.astype(jnp.int32),
                       domain_id.astype(jnp.int32), u, emb)
    return out[:BATCH]


# speedup vs baseline: 1.1825x; 1.1825x over previous
import functools

import jax
import jax.numpy as jnp
from jax import lax
from jax.experimental import pallas as pl
from jax.experimental.pallas import tpu as pltpu
from jax.experimental.pallas import tpu_sc as plsc

NUM_STYLES = 100000
DIM = 64
BATCH = 16384
P_DROP = 0.5

_info = plsc.get_sparse_core_info()
_NC, _NS, _L = _info.num_cores, _info.num_subcores, _info.num_lanes
_NW = _NC * _NS
_BPW = BATCH // _NW
_GCH = 128
_NG = _BPW // _GCH
_HALF = 50048

_mesh = plsc.VectorSubcoreMesh(core_axis_name="c", subcore_axis_name="s")


@functools.partial(
    pl.kernel,
    mesh=_mesh,
    compiler_params=pltpu.CompilerParams(use_tc_tiling_on_sc=False,
                                         needs_layout_passes=False),
    out_type=jax.ShapeDtypeStruct((DIM, BATCH), jnp.float32),
    scratch_types=[
        pltpu.VMEM((_NG, _GCH), jnp.int32),
        pltpu.VMEM((_BPW,), jnp.int32),
        pltpu.VMEM((_BPW,), jnp.int32),
        pltpu.VMEM((_BPW,), jnp.int32),
        pltpu.VMEM((_BPW,), jnp.float32),
        pltpu.VMEM((_BPW + 1, DIM), jnp.float32),
        pltpu.VMEM((DIM, _BPW), jnp.float32),
        pltpu.SemaphoreType.DMA,
        pltpu.SemaphoreType.DMA,
    ],
)
def _style_embed(style_hbm, domain_hbm, u_hbm, emb_hbm, out_hbm,
                 gid_v, fix_v, sty_v, dom_v, u_v, slab_v, slabt_v, sem, sem2):
    wid = lax.axis_index("s") * _NC + lax.axis_index("c")
    base = wid * _BPW
    stages = [
        pltpu.async_copy(style_hbm.at[pl.ds(base, _BPW)], sty_v, sem2),
        pltpu.async_copy(domain_hbm.at[pl.ds(base, _BPW)], dom_v, sem2),
        pltpu.async_copy(u_hbm.at[pl.ds(base, _BPW)], u_v, sem2),
    ]
    for st in stages:
        st.wait()
    iota = lax.iota(jnp.int32, _L)
    for g in range(_BPW // _L):
        off = g * _L
        s = sty_v[pl.ds(off, _L)]
        d = dom_v[pl.ds(off, _L)]
        u = u_v[pl.ds(off, _L)]
        gid_v[off // _GCH, pl.ds(off % _GCH, _L)] = jnp.where(
            s < _HALF, 2 * s, 2 * (s - _HALF) + 1)
        drop = (d == 1) & (u < P_DROP)
        fix_v[pl.ds(off, _L)] = jnp.where(drop, _BPW, iota + off)
    gathers = [
        pltpu.async_copy(emb_hbm.at[gid_v.at[j]],
                         slab_v.at[pl.ds(j * _GCH, _GCH)], sem)
        for j in range(_NG)
    ]
    pltpu.sync_copy(emb_hbm.at[pl.ds(0, 1)], slab_v.at[pl.ds(_BPW, 1)])
    for cp in gathers:
        cp.wait()
    for g in range(_BPW // _L):
        rows = fix_v[pl.ds(g * _L, _L)]
        zero = iota * 0
        for f in range(DIM):
            slabt_v[f, pl.ds(g * _L, _L)] = plsc.load_gather(
                slab_v, [rows, zero + f])
    pltpu.sync_copy(slabt_v, out_hbm.at[:, pl.ds(base, _BPW)])


_RB = 2176
_RGRID = _HALF // _RB


def _mxu_transpose(x, eye_bf):
    dn = (((0,), (0,)), ((), ()))
    a = x.astype(jnp.bfloat16)
    b = (x - a.astype(jnp.float32)).astype(jnp.bfloat16)
    t = lax.dot_general(a, eye_bf, dn, preferred_element_type=jnp.float32)
    t += lax.dot_general(b, eye_bf, dn, preferred_element_type=jnp.float32)
    return t


def _repack_body(lo_ref, hi_ref, out_ref):
    eye_bf = (lax.broadcasted_iota(jnp.int32, (DIM, DIM), 0) ==
              lax.broadcasted_iota(jnp.int32, (DIM, DIM), 1)
              ).astype(jnp.bfloat16)
    out_ref[...] = jnp.concatenate([_mxu_transpose(lo_ref[...], eye_bf),
                                    _mxu_transpose(hi_ref[...], eye_bf)],
                                   axis=1)


_repack = pl.pallas_call(
    _repack_body,
    grid=(_RGRID,),
    in_specs=[pl.BlockSpec((DIM, _RB), lambda b: (0, b)),
              pl.BlockSpec((DIM, _RB), lambda b: (0, b + _RGRID))],
    out_specs=pl.BlockSpec((_RB, 2 * DIM), lambda b: (b, 0)),
    out_shape=jax.ShapeDtypeStruct((_HALF, 2 * DIM), jnp.float32),
)


def kernel(style_id, domain_id, emb):
    u = jax.random.uniform(jax.random.key(42), style_id.shape, dtype=jnp.float32)
    emb_t = emb.T
    table = _repack(emb_t, emb_t).reshape(2 * _HALF, DIM)
    out_t = _style_embed(style_id.astype(jnp.int32),
                         domain_id.astype(jnp.int32), u, table)
    return out_t.T


# speedup vs baseline: 1.3746x; 1.1625x over previous
import functools

import jax
import jax.numpy as jnp
from jax import lax
from jax.experimental import pallas as pl
from jax.experimental.pallas import tpu as pltpu
from jax.experimental.pallas import tpu_sc as plsc

NUM_STYLES = 100000
DIM = 64
BATCH = 16384
P_DROP = 0.5

_info = plsc.get_sparse_core_info()
_NC, _NS, _L = _info.num_cores, _info.num_subcores, _info.num_lanes
_NW = _NC * _NS
_BPW = BATCH // _NW
_GCH = 128
_NG = _BPW // _GCH
_NTRASH = 512
_HALF = 50048

_mesh = plsc.VectorSubcoreMesh(core_axis_name="c", subcore_axis_name="s")


@functools.partial(
    pl.kernel,
    mesh=_mesh,
    compiler_params=pltpu.CompilerParams(use_tc_tiling_on_sc=False),
    out_type=jax.ShapeDtypeStruct((BATCH + _NTRASH, DIM), jnp.float32),
    scratch_types=[
        pltpu.VMEM((_NG, _GCH), jnp.int32),
        pltpu.VMEM((_NG, _GCH), jnp.int32),
        pltpu.VMEM((_BPW,), jnp.int32),
        pltpu.VMEM((_BPW,), jnp.int32),
        pltpu.VMEM((_BPW,), jnp.float32),
        pltpu.VMEM((_GCH, DIM), jnp.float32),
        pltpu.VMEM((_BPW, DIM), jnp.float32),
        pltpu.SemaphoreType.DMA,
        pltpu.SemaphoreType.DMA,
    ],
)
def _style_embed(style_hbm, domain_hbm, u_hbm, emb_hbm, out_hbm,
                 gid_v, fix_v, sty_v, dom_v, u_v, rep0_v, slab_v, sem, sem2):
    wid = lax.axis_index("s") * _NC + lax.axis_index("c")
    base = wid * _BPW
    stages = [
        pltpu.async_copy(style_hbm.at[pl.ds(base, _BPW)], sty_v, sem2),
        pltpu.async_copy(domain_hbm.at[pl.ds(base, _BPW)], dom_v, sem2),
        pltpu.async_copy(u_hbm.at[pl.ds(base, _BPW)], u_v, sem2),
    ]
    for st in stages:
        st.wait()
    iota = lax.iota(jnp.int32, _L)
    for g in range(_BPW // _L):
        off = g * _L
        s = sty_v[pl.ds(off, _L)]
        d = dom_v[pl.ds(off, _L)]
        u = u_v[pl.ds(off, _L)]
        gid_v[off // _GCH, pl.ds(off % _GCH, _L)] = jnp.where(
            s < _HALF, 2 * s, 2 * (s - _HALF) + 1)
        drop = (d == 1) & (u < P_DROP)
        lane = iota + off
        trash = BATCH + ((base + off) % _NTRASH) + iota
        fix_v[off // _GCH, pl.ds(off % _GCH, _L)] = jnp.where(
            drop, base + lane, trash)
    gathers = [
        pltpu.async_copy(emb_hbm.at[gid_v.at[j]],
                         slab_v.at[pl.ds(j * _GCH, _GCH)], sem)
        for j in range(_NG)
    ]
    pltpu.sync_copy(emb_hbm.at[pl.ds(0, 1)], rep0_v.at[pl.ds(0, 1)])
    r0 = [rep0_v[0, pl.ds(c * _L, _L)] for c in range(DIM // _L)]
    for r in range(1, _GCH):
        for c in range(DIM // _L):
            rep0_v[r, pl.ds(c * _L, _L)] = r0[c]
    for cp in gathers:
        cp.wait()
    pltpu.sync_copy(slab_v, out_hbm.at[pl.ds(base, _BPW)])
    fixups = [
        pltpu.async_copy(rep0_v, out_hbm.at[fix_v.at[j]], sem)
        for j in range(_NG)
    ]
    for cp in fixups:
        cp.wait()


_RB = 2176
_RGRID = _HALF // _RB


def _mxu_transpose(x, eye_bf):
    dn = (((0,), (0,)), ((), ()))
    a = x.astype(jnp.bfloat16)
    b = (x - a.astype(jnp.float32)).astype(jnp.bfloat16)
    t = lax.dot_general(a, eye_bf, dn, preferred_element_type=jnp.float32)
    t += lax.dot_general(b, eye_bf, dn, preferred_element_type=jnp.float32)
    return t


def _repack_body(lo_ref, hi_ref, out_ref):
    eye_bf = (lax.broadcasted_iota(jnp.int32, (DIM, DIM), 0) ==
              lax.broadcasted_iota(jnp.int32, (DIM, DIM), 1)
              ).astype(jnp.bfloat16)
    out_ref[...] = jnp.concatenate([_mxu_transpose(lo_ref[...], eye_bf),
                                    _mxu_transpose(hi_ref[...], eye_bf)],
                                   axis=1)


_repack = pl.pallas_call(
    _repack_body,
    grid=(_RGRID,),
    in_specs=[pl.BlockSpec((DIM, _RB), lambda b: (0, b)),
              pl.BlockSpec((DIM, _RB), lambda b: (0, b + _RGRID))],
    out_specs=pl.BlockSpec((_RB, 2 * DIM), lambda b: (b, 0)),
    out_shape=jax.ShapeDtypeStruct((_HALF, 2 * DIM), jnp.float32),
)


def kernel(style_id, domain_id, emb):
    u = jax.random.uniform(jax.random.key(42), style_id.shape, dtype=jnp.float32)
    emb_t = emb.T
    table = _repack(emb_t, emb_t).reshape(2 * _HALF, DIM)
    out = _style_embed(style_id.astype(jnp.int32),
                       domain_id.astype(jnp.int32), u, table)
    return out[:BATCH]
